# Initial kernel scaffold; baseline (speedup 1.0000x reference)
import jax
import jax.numpy as jnp
from jax.experimental import pallas as pl


def kernel(xyz, parent_xyz, feats):
    raise NotImplementedError("write your pallas kernel here")


# speedup vs baseline: 33.5874x; 33.5874x over previous
import functools

import jax
import jax.numpy as jnp
from jax.experimental import pallas as pl

_NB = 512


def _block_kernel(xyz_ref, pt_ref, feats_ref, out_ref):
    x = xyz_ref[...]
    p = pt_ref[...]
    m = x.shape[0]
    nb = p.shape[1]
    d2 = (
        (x[:, 0:1] - p[0:1, :]) ** 2
        + (x[:, 1:2] - p[1:2, :]) ** 2
        + (x[:, 2:3] - p[2:3, :]) ** 2
    )
    iota = jax.lax.broadcasted_iota(jnp.int32, (m, nb), 0)
    inf = jnp.float32(jnp.inf)
    wt = jnp.zeros((m, nb), jnp.float32)
    norm = jnp.zeros((1, nb), jnp.float32)
    for _ in range(3):
        mv = jnp.min(d2, axis=0, keepdims=True)
        eq = d2 == mv
        mi = jnp.min(jnp.where(eq, iota, m), axis=0, keepdims=True)
        sel = iota == mi
        d2 = jnp.where(sel, inf, d2)
        inv = 1.0 / (mv + 1e-8)
        norm = norm + inv
        wt = wt + jnp.where(sel, inv, 0.0)
    wt = wt / norm
    out_ref[...] = jnp.dot(
        feats_ref[...], wt, preferred_element_type=jnp.float32
    )


@jax.jit
def kernel(xyz, parent_xyz, feats):
    bs, m, _ = xyz.shape
    n = parent_xyz.shape[1]
    c = feats.shape[1]
    parent_t = jnp.transpose(parent_xyz, (0, 2, 1))
    grid = (bs, n // _NB)
    return pl.pallas_call(
        _block_kernel,
        grid=grid,
        in_specs=[
            pl.BlockSpec((None, m, 3), lambda b, i: (b, 0, 0)),
            pl.BlockSpec((None, 3, _NB), lambda b, i: (b, 0, i)),
            pl.BlockSpec((None, c, m), lambda b, i: (b, 0, 0)),
        ],
        out_specs=pl.BlockSpec((None, c, _NB), lambda b, i: (b, 0, i)),
        out_shape=jax.ShapeDtypeStruct((bs, c, n), jnp.float32),
    )(xyz, parent_t, feats)


# speedup vs baseline: 60.7686x; 1.8093x over previous
import jax
import jax.numpy as jnp
from jax.experimental import pallas as pl

_NB = 512


def _block_kernel(xyz_ref, pt_ref, feats_ref, out_ref):
    x = xyz_ref[...]
    p = pt_ref[...]
    m = x.shape[0]
    nb = p.shape[1]

    xp = jnp.dot(x, p, preferred_element_type=jnp.float32)
    xn = jnp.sum(x * x, axis=1, keepdims=True)
    pn = jnp.sum(p * p, axis=0, keepdims=True)
    d2 = jnp.maximum((xn + pn) - 2.0 * xp, 0.0)

    h = m // 2
    a, b = d2[:h], d2[h:]
    s1 = jnp.minimum(a, b)
    s2 = jnp.maximum(a, b)
    q = h // 2
    a1, a2 = s1[:q], s2[:q]
    b1, b2 = s1[q:], s2[q:]
    k1 = jnp.minimum(a1, b1)
    v = jnp.maximum(a1, b1)
    u = jnp.minimum(a2, b2)
    k2 = jnp.minimum(v, u)
    k3 = jnp.maximum(v, u)
    r = q // 2
    while r >= 1:
        a1, a2, a3 = k1[:r], k2[:r], k3[:r]
        b1, b2, b3 = k1[r:], k2[r:], k3[r:]
        n1 = jnp.minimum(a1, b1)
        n2 = jnp.minimum(jnp.minimum(a2, b2), jnp.maximum(a1, b1))
        n3 = jnp.minimum(
            jnp.minimum(a3, b3),
            jnp.minimum(jnp.maximum(a2, b1), jnp.maximum(a1, b2)),
        )
        k1, k2, k3 = n1, n2, n3
        r //= 2

    inv1 = 1.0 / (k1 + 1e-8)
    inv2 = 1.0 / (k2 + 1e-8)
    inv3 = 1.0 / (k3 + 1e-8)
    norm = inv1 + inv2 + inv3

    wt = (
        jnp.where(d2 == k1, inv1 / norm, 0.0)
        + jnp.where(d2 == k2, inv2 / norm, 0.0)
        + jnp.where(d2 == k3, inv3 / norm, 0.0)
    )
    out_ref[...] = jnp.dot(
        feats_ref[...], wt, preferred_element_type=jnp.float32
    )


@jax.jit
def kernel(xyz, parent_xyz, feats):
    bs, m, _ = xyz.shape
    n = parent_xyz.shape[1]
    c = feats.shape[1]
    parent_t = jnp.transpose(parent_xyz, (0, 2, 1))
    grid = (bs, n // _NB)
    return pl.pallas_call(
        _block_kernel,
        grid=grid,
        in_specs=[
            pl.BlockSpec((None, m, 3), lambda b, i: (b, 0, 0)),
            pl.BlockSpec((None, 3, _NB), lambda b, i: (b, 0, i)),
            pl.BlockSpec((None, c, m), lambda b, i: (b, 0, 0)),
        ],
        out_specs=pl.BlockSpec((None, c, _NB), lambda b, i: (b, 0, i)),
        out_shape=jax.ShapeDtypeStruct((bs, c, n), jnp.float32),
    )(xyz, parent_t, feats)
